# baseline (speedup 1.0000x reference)
import jax
import jax.numpy as jnp
from jax import lax
from jax.experimental import pallas as pl
from jax.experimental.pallas import tpu as pltpu
from jax.experimental.pallas import tpu_sc as plsc

B = 16384
D = 64
NC = 2
NS = 16
NW = NC * NS
BPW = B // NW
L = 16
NV = 100000
NP = NV // 2

PPW = 1568
PCH = 128
PCHUNKS = [PCH] * 12 + [PPW - 12 * PCH]

GCH = 64
NCH = BPW // GCH


def _depad_body(t0, t1, t2, t3, o0, o1, o2, o3, vin, vout, isem, osem):
    wid = lax.axis_index("s") * NC + lax.axis_index("c")
    start = jnp.minimum(wid * PPW, NP - PPW)

    jobs = []
    for t, o in ((t0, o0), (t1, o1), (t2, o2), (t3, o3)):
        off = 0
        for c in PCHUNKS:
            jobs.append((t, o, off, c))
            off += c

    n = len(jobs)
    ind = [None] * n
    outd = [None] * n

    def issue(k):
        t, _, off, c = jobs[k]
        b = k % 2
        ind[k] = pltpu.async_copy(
            t.at[pl.ds((start + off) * 2, c * 2)],
            vin.at[b, pl.ds(0, c * 2)], isem)

    def pack(k):
        b = k % 2
        c = jobs[k][3]

        def row(p, _):
            for half in range(2):
                for piece in range(D // L):
                    vout.at[b][p, pl.ds(half * D + piece * L, L)] = (
                        vin.at[b][2 * p + half, pl.ds(piece * L, L)])
            return 0

        lax.fori_loop(0, c, row, 0)

    issue(0)
    for k in range(n):
        b = k % 2
        _, o, off, c = jobs[k]
        ind[k].wait()
        if k + 1 < n:
            if k - 1 >= 0:
                outd[k - 1].wait()
            issue(k + 1)
        pack(k)
        outd[k] = pltpu.async_copy(
            vout.at[b, pl.ds(0, c)], o.at[pl.ds(start + off, c)], osem)
    outd[n - 2].wait()
    outd[n - 1].wait()


@jax.jit
def _sc_depad(u_mlp, i_mlp, u_gmf, i_gmf):
    mesh = plsc.VectorSubcoreMesh(core_axis_name="c", subcore_axis_name="s")
    out = jax.ShapeDtypeStruct((NP, 2 * D), jnp.float32)
    f = pl.kernel(
        _depad_body,
        out_type=(out, out, out, out),
        mesh=mesh,
        scratch_types=[
            pltpu.VMEM((2, 2 * PCH, D), jnp.float32),
            pltpu.VMEM((2, PCH, 2 * D), jnp.float32),
            pltpu.SemaphoreType.DMA,
            pltpu.SemaphoreType.DMA,
        ],
        compiler_params=pltpu.CompilerParams(
            use_tc_tiling_on_sc=True, needs_layout_passes=False),
    )
    return f(u_mlp, i_mlp, u_gmf, i_gmf)


def _sc_gather_body(uid_hbm, iid_hbm, u_mlp_p, i_mlp_p, u_gmf_p, i_gmf_p,
                    out_mlp, out_gmf,
                    idx_u, idx_i, pidx, pcol, pbuf, rowbuf, gsem, wsem):
    wid = lax.axis_index("s") * NC + lax.axis_index("c")
    base = wid * BPW

    for j in range(NCH):
        pltpu.sync_copy(uid_hbm.at[pl.ds(base + j * GCH, GCH)], idx_u.at[j])
        pltpu.sync_copy(iid_hbm.at[pl.ds(base + j * GCH, GCH)], idx_i.at[j])

    for s, idx in enumerate((idx_u, idx_i)):
        for j in range(NCH):
            for g in range(GCH // L):
                v = idx[j, pl.ds(g * L, L)]
                pidx[s * NCH + j, pl.ds(g * L, L)] = v >> 1
                pcol[s * NCH + j, pl.ds(g * L, L)] = D * (v & 1)

    steps = []
    for ut, it, out in ((u_mlp_p, i_mlp_p, out_mlp),
                        (u_gmf_p, i_gmf_p, out_gmf)):
        for j in range(NCH):
            steps.append((ut, it, out, j))

    n = len(steps)
    gd = [None] * n
    wd = [None] * n

    def start_gather(k):
        ut, it, _, j = steps[k]
        b = k % 2
        gd[k] = (
            pltpu.async_copy(ut.at[pidx.at[j]], pbuf.at[b, 0], gsem),
            pltpu.async_copy(it.at[pidx.at[NCH + j]], pbuf.at[b, 1], gsem),
        )

    iota = lax.iota(jnp.int32, L)

    def select_rows(k):
        b = k % 2
        _, _, _, j = steps[k]
        rb = rowbuf.at[b]
        for s in range(2):
            pb = pbuf.at[b, s]
            sj = s * NCH + j

            def group(g, _):
                rows16 = g * L + iota
                cbase = pcol[sj, pl.ds(g * L, L)]

                def colblk(c4, _):
                    for u in range(4):
                        c = c4 * 4 + u
                        vals = plsc.load_gather(pb, [rows16, cbase + c])
                        plsc.store_scatter(
                            rb, [rows16,
                                 jnp.full((L,), s * D, jnp.int32) + c], vals)
                    return 0

                lax.fori_loop(0, D // 4, colblk, 0)
                return 0

            lax.fori_loop(0, GCH // L, group, 0)

    start_gather(0)
    for k in range(n):
        b = k % 2
        _, _, out, j = steps[k]
        gd[k][0].wait()
        gd[k][1].wait()
        if k + 1 < n:
            start_gather(k + 1)
        if k >= 2:
            wd[k - 2].wait()
        select_rows(k)
        wd[k] = pltpu.async_copy(
            rowbuf.at[b], out.at[pl.ds(base + j * GCH, GCH)], wsem)
    wd[n - 2].wait()
    wd[n - 1].wait()


@jax.jit
def _sc_gather(user_id, item_id, u_mlp_p, i_mlp_p, u_gmf_p, i_gmf_p):
    mesh = plsc.VectorSubcoreMesh(core_axis_name="c", subcore_axis_name="s")
    out = jax.ShapeDtypeStruct((B, 2 * D), jnp.float32)
    f = pl.kernel(
        _sc_gather_body,
        out_type=(out, out),
        mesh=mesh,
        scratch_types=[
            pltpu.VMEM((NCH, GCH), jnp.int32),
            pltpu.VMEM((NCH, GCH), jnp.int32),
            pltpu.VMEM((2 * NCH, GCH), jnp.int32),
            pltpu.VMEM((2 * NCH, GCH), jnp.int32),
            pltpu.VMEM((2, 2, GCH, 2 * D), jnp.float32),
            pltpu.VMEM((2, GCH, 2 * D), jnp.float32),
            pltpu.SemaphoreType.DMA,
            pltpu.SemaphoreType.DMA,
        ],
        compiler_params=pltpu.CompilerParams(
            use_tc_tiling_on_sc=True, needs_layout_passes=False),
    )
    return f(user_id, item_id, u_mlp_p, i_mlp_p, u_gmf_p, i_gmf_p)


def _mlp_body(x1_ref, x2_ref, w0_ref, b0_ref, w1_ref, b1_ref,
              w2_ref, b2_ref, wp_ref, out_ref):
    h = jnp.maximum(
        jnp.dot(x1_ref[...], w0_ref[...], preferred_element_type=jnp.float32)
        + b0_ref[...], 0.0)
    h = jnp.maximum(
        jnp.dot(h, w1_ref[...], preferred_element_type=jnp.float32)
        + b1_ref[...], 0.0)
    h = jnp.maximum(
        jnp.dot(h, w2_ref[...], preferred_element_type=jnp.float32)
        + b2_ref[...], 0.0)
    x2 = x2_ref[...]
    gmf = x2[:, :D] * x2[:, D:]
    out_ref[...] = (
        jnp.dot(gmf, wp_ref[0:D, :], preferred_element_type=jnp.float32)
        + jnp.dot(h, wp_ref[D:2 * D, :], preferred_element_type=jnp.float32))


BM = 2048


def _tc_mlp(x1, x2, W0, b0, W1, b1, W2, b2, Wp, interpret=False):
    grid = (B // BM,)
    row_spec = pl.BlockSpec((BM, 2 * D), lambda i: (i, 0))
    full = lambda shape: pl.BlockSpec(shape, lambda i: tuple(0 for _ in shape))
    return pl.pallas_call(
        _mlp_body,
        grid=grid,
        in_specs=[
            row_spec, row_spec,
            full(W0.shape), full((1, 256)),
            full(W1.shape), full((1, 128)),
            full(W2.shape), full((1, 64)),
            full((128, 1)),
        ],
        out_specs=pl.BlockSpec((BM, 1), lambda i: (i, 0)),
        out_shape=jax.ShapeDtypeStruct((B, 1), jnp.float32),
        interpret=interpret,
    )(x1, x2, W0, b0.reshape(1, -1), W1, b1.reshape(1, -1),
      W2, b2.reshape(1, -1), Wp)


def kernel(user_id, item_id, u_mlp, i_mlp, u_gmf, i_gmf,
           W0, b0, W1, b1, W2, b2, Wp):
    um_p, im_p, ug_p, ig_p = _sc_depad(u_mlp, i_mlp, u_gmf, i_gmf)
    x1, x2 = _sc_gather(user_id, item_id, um_p, im_p, ug_p, ig_p)
    return _tc_mlp(x1, x2, W0, b0, W1, b1, W2, b2, Wp).reshape(-1)


# speedup vs baseline: 2.3142x; 2.3142x over previous
import jax
import jax.numpy as jnp
from jax import lax
from jax.experimental import pallas as pl
from jax.experimental.pallas import tpu as pltpu
from jax.experimental.pallas import tpu_sc as plsc

B = 16384
D = 64
NC = 2
NS = 16
NW = NC * NS
BPW = B // NW
CHUNK = 128
NCHUNK = BPW // CHUNK
NV = 100000


def _sc_gather_body(uid_hbm, iid_hbm, cu_hbm, ci_hbm, out_u, out_i,
                    idx_u, idx_i, rows, sem0, sem1, wsem):
    wid = lax.axis_index("s") * NC + lax.axis_index("c")
    base = wid * BPW

    for j in range(NCHUNK):
        pltpu.sync_copy(uid_hbm.at[pl.ds(base + j * CHUNK, CHUNK)], idx_u.at[j])
        pltpu.sync_copy(iid_hbm.at[pl.ds(base + j * CHUNK, CHUNK)], idx_i.at[j])

    steps = []
    for tbl, idx, out in ((cu_hbm, idx_u, out_u), (ci_hbm, idx_i, out_i)):
        for j in range(NCHUNK):
            steps.append((tbl, idx, out, j))

    sems = (sem0, sem1)
    n = len(steps)
    descs = [None] * n
    wd = [None] * n

    tbl0, i0, _, j0 = steps[0]
    descs[0] = pltpu.async_copy(tbl0.at[i0.at[j0]], rows.at[0], sems[0])
    for k in range(n):
        buf = k % 2
        if k + 1 < n:
            tbl, idx, _, j = steps[k + 1]
            if k - 1 >= 0:
                wd[k - 1].wait()
            descs[k + 1] = pltpu.async_copy(
                tbl.at[idx.at[j]], rows.at[1 - buf], sems[1 - buf])
        descs[k].wait()
        _, _, out, j = steps[k]
        wd[k] = pltpu.async_copy(
            rows.at[buf], out.at[pl.ds(base + j * CHUNK, CHUNK)], wsem)
    wd[n - 2].wait()
    wd[n - 1].wait()


@jax.jit
def _sc_gather(user_id, item_id, cu, ci):
    mesh = plsc.VectorSubcoreMesh(core_axis_name="c", subcore_axis_name="s")
    out = jax.ShapeDtypeStruct((B, 2 * D), jnp.float32)
    f = pl.kernel(
        _sc_gather_body,
        out_type=(out, out),
        mesh=mesh,
        scratch_types=[
            pltpu.VMEM((NCHUNK, CHUNK), jnp.int32),
            pltpu.VMEM((NCHUNK, CHUNK), jnp.int32),
            pltpu.VMEM((2, CHUNK, 2 * D), jnp.float32),
            pltpu.SemaphoreType.DMA,
            pltpu.SemaphoreType.DMA,
            pltpu.SemaphoreType.DMA,
        ],
        compiler_params=pltpu.CompilerParams(use_tc_tiling_on_sc=True),
    )
    return f(user_id, item_id, cu, ci)


def _mlp_body(xu_ref, xi_ref, w0_ref, b0_ref, w1_ref, b1_ref,
              w2_ref, b2_ref, wp_ref, out_ref):
    xu = xu_ref[...]
    xi = xi_ref[...]
    h = jnp.concatenate([xu[:, :D], xi[:, :D]], axis=1)
    h = jnp.maximum(
        jnp.dot(h, w0_ref[...], preferred_element_type=jnp.float32)
        + b0_ref[...], 0.0)
    h = jnp.maximum(
        jnp.dot(h, w1_ref[...], preferred_element_type=jnp.float32)
        + b1_ref[...], 0.0)
    h = jnp.maximum(
        jnp.dot(h, w2_ref[...], preferred_element_type=jnp.float32)
        + b2_ref[...], 0.0)
    gmf = xu[:, D:] * xi[:, D:]
    out_ref[...] = (
        jnp.dot(gmf, wp_ref[0:D, :], preferred_element_type=jnp.float32)
        + jnp.dot(h, wp_ref[D:2 * D, :], preferred_element_type=jnp.float32))


BM = 2048


def _tc_mlp(xu, xi, W0, b0, W1, b1, W2, b2, Wp, interpret=False):
    grid = (B // BM,)
    row_spec = pl.BlockSpec((BM, 2 * D), lambda i: (i, 0))
    full = lambda shape: pl.BlockSpec(shape, lambda i: tuple(0 for _ in shape))
    return pl.pallas_call(
        _mlp_body,
        grid=grid,
        in_specs=[
            row_spec, row_spec,
            full(W0.shape), full((1, 256)),
            full(W1.shape), full((1, 128)),
            full(W2.shape), full((1, 64)),
            full((128, 1)),
        ],
        out_specs=pl.BlockSpec((BM, 1), lambda i: (i, 0)),
        out_shape=jax.ShapeDtypeStruct((B, 1), jnp.float32),
        interpret=interpret,
    )(xu, xi, W0, b0.reshape(1, -1), W1, b1.reshape(1, -1),
      W2, b2.reshape(1, -1), Wp)


def kernel(user_id, item_id, u_mlp, i_mlp, u_gmf, i_gmf,
           W0, b0, W1, b1, W2, b2, Wp):
    cu = jnp.concatenate([u_mlp, u_gmf], axis=1)
    ci = jnp.concatenate([i_mlp, i_gmf], axis=1)
    xu, xi = _sc_gather(user_id, item_id, cu, ci)
    return _tc_mlp(xu, xi, W0, b0, W1, b1, W2, b2, Wp).reshape(-1)
